# baseline (speedup 1.0000x reference)
import functools

import jax
import jax.numpy as jnp
from jax import lax
from jax.experimental import pallas as pl
from jax.experimental.pallas import tpu as pltpu
from jax.experimental.pallas import tpu_sc as plsc

CS = 16
LANES = 16


@functools.lru_cache(maxsize=None)
def _build(B, S, D, V):
    mesh = plsc.VectorSubcoreMesh(core_axis_name="c", subcore_axis_name="s")
    NC, NS = mesh.num_cores, mesh.num_subcores
    NW = NC * NS
    assert S % (NW * CS) == 0 and D % LANES == 0
    S_PER_W = S // NW
    NCHUNK = S_PER_W // CS
    assert NCHUNK >= 2 and NCHUNK % 2 == 0 and B == 4

    @functools.partial(
        pl.kernel,
        out_type=jax.ShapeDtypeStruct((B * S, D), jnp.float32),
        mesh=mesh,
        scratch_types=[
            pltpu.VMEM((B, CS), jnp.int32),
            pltpu.VMEM((B, CS), jnp.int32),
            pltpu.VMEM((CS, D), jnp.float32),
            pltpu.VMEM((CS, D), jnp.float32),
            pltpu.VMEM((CS, D), jnp.float32),
            pltpu.VMEM((CS, D), jnp.float32),
            pltpu.VMEM((CS, D), jnp.float32),
            pltpu.VMEM((CS, D), jnp.float32),
        ] + [pltpu.SemaphoreType.DMA] * 12,
    )
    def emb(ids_hbm, tok_hbm, pos_hbm, out_hbm, x0_v, x1_v,
            r0_v, r1_v, r2_v, r3_v, p0_v, p1_v,
            g0, g1, g2, g3, o0, o1, o2, o3, ps0, ps1, is0, is1):
        idx = (x0_v, x1_v)
        rows = (r0_v, r1_v, r2_v, r3_v)
        pos = (p0_v, p1_v)
        gsem = (g0, g1, g2, g3)
        osem = (o0, o1, o2, o3)
        psem = (ps0, ps1)
        isem = (is0, is1)

        wid = lax.axis_index("s") * NC + lax.axis_index("c")
        s_base = wid * S_PER_W
        smax = S - CS

        def start_idx(i_c, hp):
            c = jnp.minimum(wid * NCHUNK + i_c, S // CS - 1)
            pltpu.async_copy(ids_hbm.at[:, c, :], idx[hp], isem[hp])

        def wait_idx(hp):
            pltpu.make_async_copy(
                ids_hbm.at[:, 0, :], idx[hp], isem[hp]).wait()

        def start_gather(hp, b, tb):
            pltpu.async_copy(tok_hbm.at[idx[hp].at[b]], rows[tb], gsem[tb])

        def start_pos(i_c, h):
            s0 = jnp.minimum(s_base + i_c * CS, smax)
            pltpu.async_copy(pos_hbm.at[pl.ds(s0, CS)], pos[h], psem[h])

        def add_and_out(i_c, b, h):
            rb = rows[b]
            ph = pos[h]

            def add_body(r, _):
                for j in range(D // LANES):
                    plsc.addupdate(rb.at[r, pl.ds(j * LANES, LANES)],
                                   ph[r, pl.ds(j * LANES, LANES)])
                return 0

            lax.fori_loop(0, CS, add_body, 0)
            r0 = b * S + s_base + i_c * CS
            pltpu.async_copy(rb, out_hbm.at[pl.ds(r0, CS)], osem[b])

        def chunk(i_c, h, guard):
            start_pos(i_c + 1, 1 - h)
            start_idx(i_c + 1, 1 - h)
            pltpu.make_async_copy(pos_hbm.at[pl.ds(0, CS)], pos[h], psem[h]).wait()
            for b in range(B):
                tb = (b + 2) % 4

                def _drain_out():
                    pltpu.make_async_copy(
                        rows[tb], out_hbm.at[pl.ds(0, CS)], osem[tb]).wait()

                if guard is not None and b < 2:
                    pl.when(guard)(_drain_out)
                else:
                    _drain_out()
                if b == 2:
                    wait_idx(1 - h)
                start_gather(h if b < 2 else 1 - h, tb, tb)
                pltpu.make_async_copy(
                    tok_hbm.at[idx[0].at[0]], rows[b], gsem[b]).wait()
                add_and_out(i_c, b, h)

        start_idx(0, 0)
        start_pos(0, 0)
        wait_idx(0)
        start_gather(0, 0, 0)
        start_gather(0, 1, 1)

        def pair_body(i2, _):
            i_c = 2 * i2
            chunk(i_c, 0, i2 > 0)
            chunk(i_c + 1, 1, None)
            return 0

        lax.fori_loop(0, NCHUNK // 2, pair_body, 0)

        pltpu.make_async_copy(tok_hbm.at[idx[0].at[0]], rows[0], gsem[0]).wait()
        pltpu.make_async_copy(tok_hbm.at[idx[0].at[1]], rows[1], gsem[1]).wait()
        pltpu.make_async_copy(rows[2], out_hbm.at[pl.ds(0, CS)], osem[2]).wait()
        pltpu.make_async_copy(rows[3], out_hbm.at[pl.ds(0, CS)], osem[3]).wait()
        pltpu.make_async_copy(pos_hbm.at[pl.ds(0, CS)], pos[0], psem[0]).wait()

    return emb


def kernel(input_ids, token_table, position_table):
    B, S = input_ids.shape
    V, D = token_table.shape
    ids_3d = input_ids.astype(jnp.int32).reshape(B, S // CS, CS)
    emb = _build(B, S, D, V)
    out = emb(ids_3d, token_table, position_table)
    return out.reshape(B, S, D)


# speedup vs baseline: 2.0432x; 2.0432x over previous
import functools

import jax
import jax.numpy as jnp
from jax import lax
from jax.experimental import pallas as pl
from jax.experimental.pallas import tpu as pltpu
from jax.experimental.pallas import tpu_sc as plsc

CS = 16
LANES = 16


@functools.lru_cache(maxsize=None)
def _build(B, S, D, V):
    mesh = plsc.VectorSubcoreMesh(core_axis_name="c", subcore_axis_name="s")
    NC, NS = mesh.num_cores, mesh.num_subcores
    NW = NC * NS
    assert S % (NW * CS) == 0 and D % LANES == 0
    S_PER_W = S // NW
    NCHUNK = S_PER_W // CS
    assert NCHUNK >= 2 and NCHUNK % 2 == 0 and B == 4

    @functools.partial(
        pl.kernel,
        out_type=jax.ShapeDtypeStruct((B * S, D), jnp.float32),
        mesh=mesh,
        scratch_types=[
            pltpu.VMEM((B * CS,), jnp.int32),
            pltpu.VMEM((B * CS,), jnp.int32),
            pltpu.VMEM((CS, D), jnp.float32),
            pltpu.VMEM((CS, D), jnp.float32),
            pltpu.VMEM((CS, D), jnp.float32),
            pltpu.VMEM((CS, D), jnp.float32),
            pltpu.VMEM((CS, D), jnp.float32),
            pltpu.VMEM((CS, D), jnp.float32),
        ] + [pltpu.SemaphoreType.DMA] * 12,
    )
    def emb(ids_hbm, tok_hbm, pos_hbm, out_hbm, x0_v, x1_v,
            r0_v, r1_v, r2_v, r3_v, p0_v, p1_v,
            g0, g1, g2, g3, o0, o1, o2, o3, ps0, ps1, is0, is1):
        idx = (x0_v, x1_v)
        rows = (r0_v, r1_v, r2_v, r3_v)
        pos = (p0_v, p1_v)
        gsem = (g0, g1, g2, g3)
        osem = (o0, o1, o2, o3)
        psem = (ps0, ps1)
        isem = (is0, is1)

        wid = lax.axis_index("s") * NC + lax.axis_index("c")
        s_base = wid * S_PER_W
        smax = S - CS

        def start_idx(i_c, hp):
            base = (wid * NCHUNK + jnp.minimum(i_c, NCHUNK - 1)) * (B * CS)
            pltpu.async_copy(ids_hbm.at[pl.ds(base, B * CS)], idx[hp], isem[hp])

        def wait_idx(hp):
            pltpu.make_async_copy(
                ids_hbm.at[pl.ds(0, B * CS)], idx[hp], isem[hp]).wait()

        def start_gather(hp, b, tb):
            pltpu.async_copy(
                tok_hbm.at[idx[hp].at[pl.ds(b * CS, CS)]], rows[tb], gsem[tb])

        def start_pos(i_c, h):
            s0 = jnp.minimum(s_base + i_c * CS, smax)
            pltpu.async_copy(pos_hbm.at[pl.ds(s0, CS)], pos[h], psem[h])

        def add_and_out(i_c, b, h):
            rb = rows[b]
            ph = pos[h]

            def add_body(r, _):
                for j in range(D // LANES):
                    plsc.addupdate(rb.at[r, pl.ds(j * LANES, LANES)],
                                   ph[r, pl.ds(j * LANES, LANES)])
                return 0

            lax.fori_loop(0, CS, add_body, 0)
            r0 = b * S + s_base + i_c * CS
            pltpu.async_copy(rb, out_hbm.at[pl.ds(r0, CS)], osem[b])

        def chunk(i_c, h, guard):
            start_pos(i_c + 1, 1 - h)
            start_idx(i_c + 1, 1 - h)
            pltpu.make_async_copy(pos_hbm.at[pl.ds(0, CS)], pos[h], psem[h]).wait()
            for b in range(B):
                tb = (b + 2) % 4

                def _drain_out():
                    pltpu.make_async_copy(
                        rows[tb], out_hbm.at[pl.ds(0, CS)], osem[tb]).wait()

                if guard is not None and b < 2:
                    pl.when(guard)(_drain_out)
                else:
                    _drain_out()
                if b == 2:
                    wait_idx(1 - h)
                start_gather(h if b < 2 else 1 - h, tb, tb)
                pltpu.make_async_copy(
                    tok_hbm.at[idx[0].at[pl.ds(0, CS)]], rows[b],
                    gsem[b]).wait()
                add_and_out(i_c, b, h)

        start_idx(0, 0)
        start_pos(0, 0)
        wait_idx(0)
        start_gather(0, 0, 0)
        start_gather(0, 1, 1)

        def pair_body(i2, _):
            i_c = 2 * i2
            chunk(i_c, 0, i2 > 0)
            chunk(i_c + 1, 1, None)
            return 0

        lax.fori_loop(0, NCHUNK // 2, pair_body, 0)

        pltpu.make_async_copy(
            tok_hbm.at[idx[0].at[pl.ds(0, CS)]], rows[0], gsem[0]).wait()
        pltpu.make_async_copy(
            tok_hbm.at[idx[0].at[pl.ds(0, CS)]], rows[1], gsem[1]).wait()
        pltpu.make_async_copy(rows[2], out_hbm.at[pl.ds(0, CS)], osem[2]).wait()
        pltpu.make_async_copy(rows[3], out_hbm.at[pl.ds(0, CS)], osem[3]).wait()
        pltpu.make_async_copy(pos_hbm.at[pl.ds(0, CS)], pos[0], psem[0]).wait()

    NWK, NCK = NW, NCHUNK
    return emb, NWK, NCK


def kernel(input_ids, token_table, position_table):
    B, S = input_ids.shape
    V, D = token_table.shape
    emb, NW, NCHUNK = _build(B, S, D, V)
    ids_r = (input_ids.astype(jnp.int32)
             .reshape(B, NW, NCHUNK, CS)
             .transpose(1, 2, 0, 3)
             .reshape(-1))
    out = emb(ids_r, token_table, position_table)
    return out.reshape(B, S, D)
